# Initial kernel scaffold; baseline (speedup 1.0000x reference)
import jax
import jax.numpy as jnp
from jax.experimental import pallas as pl


def kernel(x, table):
    raise NotImplementedError("write your pallas kernel here")


# speedup vs baseline: 3.3422x; 3.3422x over previous
import functools

import jax
import jax.numpy as jnp
from jax import lax
from jax.experimental import pallas as pl
from jax.experimental.pallas import tpu as pltpu
from jax.experimental.pallas import tpu_sc as plsc

BATCH = 4096
SEQ = 200
DIM = 128
LANES = 16

NUM_CORES = 2
NUM_SUBCORES = 16
NUM_WORKERS = NUM_CORES * NUM_SUBCORES

ROWS = BATCH * SEQ
ROWS_PER_WORKER = ROWS // NUM_WORKERS
CHUNK = 128
CHUNKS_PER_WORKER = ROWS_PER_WORKER // CHUNK


def _positional_encoding():
    pos = jnp.arange(0, SEQ, dtype=jnp.float32)
    i = jnp.arange(0, DIM)
    i = 2 * (i // 2) / DIM
    i = 1.0 / jnp.power(10000.0, i.astype(jnp.float32))
    mat = jnp.outer(pos, i)
    mat = mat.at[:, ::2].set(jnp.sin(mat[:, ::2]))
    mat = mat.at[:, 1::2].set(jnp.cos(mat[:, 1::2]))
    return mat


@functools.partial(
    pl.kernel,
    mesh=plsc.VectorSubcoreMesh(core_axis_name="c", subcore_axis_name="s"),
    out_type=jax.ShapeDtypeStruct((ROWS, DIM), jnp.float32),
    scratch_types=[
        pltpu.VMEM((CHUNKS_PER_WORKER, CHUNK), jnp.int32),
        pltpu.VMEM((SEQ, DIM), jnp.float32),
        pltpu.VMEM((2, CHUNK, DIM), jnp.float32),
        pltpu.SemaphoreType.DMA,
        pltpu.SemaphoreType.DMA,
        pltpu.SemaphoreType.DMA,
        pltpu.SemaphoreType.DMA,
    ],
)
def _embed_kernel(x_hbm, table_hbm, pe_hbm, out_hbm,
                  idx_v, pe_v, rows_v, g0, g1, s0, s1):
    wid = lax.axis_index("s") * NUM_CORES + lax.axis_index("c")
    row_base = wid * ROWS_PER_WORKER
    gsem = (g0, g1)
    ssem = (s0, s1)

    pltpu.sync_copy(pe_hbm, pe_v)
    pltpu.sync_copy(x_hbm.at[pl.ds(wid * CHUNKS_PER_WORKER, CHUNKS_PER_WORKER)],
                    idx_v)

    def start_gather(c, buf):
        pltpu.async_copy(table_hbm.at[idx_v.at[c]], rows_v.at[buf], gsem[buf])

    def wait_gather(c, buf):
        pltpu.make_async_copy(table_hbm.at[idx_v.at[c]], rows_v.at[buf],
                              gsem[buf]).wait()

    def start_scatter(c, buf):
        pltpu.async_copy(rows_v.at[buf],
                         out_hbm.at[pl.ds(row_base + c * CHUNK, CHUNK)],
                         ssem[buf])

    def wait_scatter(c, buf):
        pltpu.make_async_copy(rows_v.at[buf],
                              out_hbm.at[pl.ds(row_base + c * CHUNK, CHUNK)],
                              ssem[buf]).wait()

    def add_pe(c, buf):
        rows = rows_v.at[buf]
        phase = lax.rem(c * CHUNK, SEQ)

        @pl.loop(0, CHUNK)
        def _row(r):
            p = phase + r
            p = jnp.where(p >= SEQ, p - SEQ, p)
            for j in range(DIM // LANES):
                sl = pl.ds(j * LANES, LANES)
                plsc.addupdate(rows.at[r, sl], pe_v[p, sl])

    start_gather(0, 0)

    @pl.loop(0, CHUNKS_PER_WORKER, step=2)
    def _outer(c0):
        for b in (0, 1):
            c = c0 + b
            wait_gather(c, b)

            @pl.when(c > 0)
            def _():
                wait_scatter(c - 1, 1 - b)

            @pl.when(c < CHUNKS_PER_WORKER - 1)
            def _():
                start_gather(c + 1, 1 - b)

            add_pe(c, b)
            start_scatter(c, b)

    wait_scatter(CHUNKS_PER_WORKER - 1, (CHUNKS_PER_WORKER - 1) % 2)


def kernel(x, table):
    pe = _positional_encoding()
    x_flat = x.astype(jnp.int32).reshape(NUM_WORKERS * CHUNKS_PER_WORKER, CHUNK)
    out = _embed_kernel(x_flat, table, pe)
    return out.reshape(BATCH, SEQ, DIM)


# speedup vs baseline: 3.4323x; 1.0270x over previous
import functools

import jax
import jax.numpy as jnp
from jax import lax
from jax.experimental import pallas as pl
from jax.experimental.pallas import tpu as pltpu
from jax.experimental.pallas import tpu_sc as plsc

BATCH = 4096
SEQ = 200
DIM = 128
LANES = 16

NUM_CORES = 2
NUM_SUBCORES = 16
NUM_WORKERS = NUM_CORES * NUM_SUBCORES

ROWS = BATCH * SEQ
ROWS_PER_WORKER = ROWS // NUM_WORKERS
CHUNK = 128
CHUNKS_PER_WORKER = ROWS_PER_WORKER // CHUNK
NBUF = 4


def _positional_encoding():
    pos = jnp.arange(0, SEQ, dtype=jnp.float32)
    i = jnp.arange(0, DIM)
    i = 2 * (i // 2) / DIM
    i = 1.0 / jnp.power(10000.0, i.astype(jnp.float32))
    mat = jnp.outer(pos, i)
    mat = mat.at[:, ::2].set(jnp.sin(mat[:, ::2]))
    mat = mat.at[:, 1::2].set(jnp.cos(mat[:, 1::2]))
    return mat


@functools.partial(
    pl.kernel,
    mesh=plsc.VectorSubcoreMesh(core_axis_name="c", subcore_axis_name="s"),
    out_type=jax.ShapeDtypeStruct((ROWS, DIM), jnp.float32),
    scratch_types=[
        pltpu.VMEM((CHUNKS_PER_WORKER, CHUNK), jnp.int32),
        pltpu.VMEM((SEQ, DIM), jnp.float32),
        pltpu.VMEM((NBUF, CHUNK, DIM), jnp.float32),
        [pltpu.SemaphoreType.DMA] * NBUF,
        [pltpu.SemaphoreType.DMA] * NBUF,
    ],
)
def _embed_kernel(x_hbm, table_hbm, pe_hbm, out_hbm,
                  idx_v, pe_v, rows_v, gsem, ssem):
    wid = lax.axis_index("s") * NUM_CORES + lax.axis_index("c")
    row_base = wid * ROWS_PER_WORKER

    pltpu.sync_copy(pe_hbm, pe_v)
    pltpu.sync_copy(x_hbm.at[pl.ds(wid * CHUNKS_PER_WORKER, CHUNKS_PER_WORKER)],
                    idx_v)

    def start_gather(c, buf):
        pltpu.async_copy(table_hbm.at[idx_v.at[c]], rows_v.at[buf], gsem[buf])

    def wait_gather(c, buf):
        pltpu.make_async_copy(table_hbm.at[idx_v.at[c]], rows_v.at[buf],
                              gsem[buf]).wait()

    def start_scatter(c, buf):
        pltpu.async_copy(rows_v.at[buf],
                         out_hbm.at[pl.ds(row_base + c * CHUNK, CHUNK)],
                         ssem[buf])

    def wait_scatter(c, buf):
        pltpu.make_async_copy(rows_v.at[buf],
                              out_hbm.at[pl.ds(row_base + c * CHUNK, CHUNK)],
                              ssem[buf]).wait()

    def add_pe(c, buf):
        rows = rows_v.at[buf]
        phase = lax.rem(c * CHUNK, SEQ)

        @pl.loop(0, CHUNK, unroll=4)
        def _row(r):
            p = phase + r
            p = jnp.where(p >= SEQ, p - SEQ, p)
            for j in range(DIM // LANES):
                sl = pl.ds(j * LANES, LANES)
                plsc.addupdate(rows.at[r, sl], pe_v[p, sl])

    for k in range(NBUF - 1):
        start_gather(k, k)

    @pl.loop(0, CHUNKS_PER_WORKER, step=NBUF)
    def _outer(c0):
        for b in range(NBUF):
            c = c0 + b
            wait_gather(c, b)

            fb = (b + NBUF - 1) % NBUF
            fc = c + NBUF - 1

            @pl.when(fc < CHUNKS_PER_WORKER)
            def _():
                @pl.when(c > 0)
                def _():
                    wait_scatter(c - 1, fb)

                start_gather(fc, fb)

            add_pe(c, b)
            start_scatter(c, b)

    for c in range(CHUNKS_PER_WORKER - NBUF, CHUNKS_PER_WORKER):
        wait_scatter(c, c % NBUF)


def kernel(x, table):
    pe = _positional_encoding()
    x_flat = x.astype(jnp.int32).reshape(NUM_WORKERS * CHUNKS_PER_WORKER, CHUNK)
    out = _embed_kernel(x_flat, table, pe)
    return out.reshape(BATCH, SEQ, DIM)


# speedup vs baseline: 9.4725x; 2.7598x over previous
import functools

import jax
import jax.numpy as jnp
from jax import lax
from jax.experimental import pallas as pl
from jax.experimental.pallas import tpu as pltpu
from jax.experimental.pallas import tpu_sc as plsc

BATCH = 4096
SEQ = 200
DIM = 128
LANES = 16

NUM_CORES = 2
NUM_SUBCORES = 16
NUM_WORKERS = NUM_CORES * NUM_SUBCORES

BPW = BATCH // NUM_WORKERS
CHUNK = BPW
NBUF = 4
ROWS = BATCH * SEQ


def _positional_encoding():
    pos = jnp.arange(0, SEQ, dtype=jnp.float32)
    i = jnp.arange(0, DIM)
    i = 2 * (i // 2) / DIM
    i = 1.0 / jnp.power(10000.0, i.astype(jnp.float32))
    mat = jnp.outer(pos, i)
    mat = mat.at[:, ::2].set(jnp.sin(mat[:, ::2]))
    mat = mat.at[:, 1::2].set(jnp.cos(mat[:, 1::2]))
    return mat


@functools.partial(
    pl.kernel,
    mesh=plsc.VectorSubcoreMesh(core_axis_name="c", subcore_axis_name="s"),
    out_type=jax.ShapeDtypeStruct((ROWS, DIM), jnp.float32),
    scratch_types=[
        pltpu.VMEM((SEQ, CHUNK), jnp.int32),
        pltpu.VMEM((SEQ, DIM), jnp.float32),
        pltpu.VMEM((NBUF, CHUNK, DIM), jnp.float32),
        pltpu.VMEM((NBUF, CHUNK), jnp.int32),
        [pltpu.SemaphoreType.DMA] * NBUF,
        [pltpu.SemaphoreType.DMA] * NBUF,
    ],
)
def _embed_kernel(xt_hbm, table_hbm, pe_hbm, out_hbm,
                  idx_v, pe_v, rows_v, didx_v, gsem, ssem):
    wid = lax.axis_index("s") * NUM_CORES + lax.axis_index("c")
    b_base = wid * BPW

    pltpu.sync_copy(pe_hbm, pe_v)
    pltpu.sync_copy(xt_hbm.at[:, pl.ds(b_base, CHUNK)], idx_v)

    ramp = [lax.iota(jnp.int32, LANES) * SEQ + (b_base + j * LANES) * SEQ
            for j in range(CHUNK // LANES)]

    def start_gather(s, buf):
        pltpu.async_copy(table_hbm.at[idx_v.at[s]], rows_v.at[buf], gsem[buf])

    def wait_gather(s, buf):
        pltpu.make_async_copy(table_hbm.at[idx_v.at[s]], rows_v.at[buf],
                              gsem[buf]).wait()

    def start_scatter(s, buf):
        for j in range(CHUNK // LANES):
            didx_v[buf, pl.ds(j * LANES, LANES)] = ramp[j] + s
        pltpu.async_copy(rows_v.at[buf], out_hbm.at[didx_v.at[buf]], ssem[buf])

    def wait_scatter(buf):
        pltpu.make_async_copy(rows_v.at[buf], out_hbm.at[didx_v.at[buf]],
                              ssem[buf]).wait()

    def add_pe(s, buf):
        rows = rows_v.at[buf]
        pe_row = [pe_v[s, pl.ds(j * LANES, LANES)] for j in range(DIM // LANES)]

        @pl.loop(0, CHUNK, unroll=4)
        def _row(r):
            for j in range(DIM // LANES):
                plsc.addupdate(rows.at[r, pl.ds(j * LANES, LANES)], pe_row[j])

    for k in range(NBUF - 1):
        start_gather(k, k)

    @pl.loop(0, SEQ, step=NBUF)
    def _outer(s0):
        for b in range(NBUF):
            s = s0 + b
            wait_gather(s, b)

            fb = (b + NBUF - 1) % NBUF
            fs = s + NBUF - 1

            @pl.when(fs < SEQ)
            def _():
                @pl.when(s > 0)
                def _():
                    wait_scatter(fb)

                start_gather(fs, fb)

            add_pe(s, b)
            start_scatter(s, b)

    for s in range(SEQ - NBUF, SEQ):
        wait_scatter(s % NBUF)


def kernel(x, table):
    pe = _positional_encoding()
    xt = x.astype(jnp.int32).T
    out = _embed_kernel(xt, table, pe)
    return out.reshape(BATCH, SEQ, DIM)


# speedup vs baseline: 9.4986x; 1.0028x over previous
import functools

import jax
import jax.numpy as jnp
from jax import lax
from jax.experimental import pallas as pl
from jax.experimental.pallas import tpu as pltpu
from jax.experimental.pallas import tpu_sc as plsc

BATCH = 4096
SEQ = 200
DIM = 128
LANES = 16

NUM_CORES = 2
NUM_SUBCORES = 16
NUM_WORKERS = NUM_CORES * NUM_SUBCORES

BPW = BATCH // NUM_WORKERS
CHUNK = BPW
NBUF = 4
ROWS = BATCH * SEQ


def _positional_encoding():
    pos = jnp.arange(0, SEQ, dtype=jnp.float32)
    i = jnp.arange(0, DIM)
    i = 2 * (i // 2) / DIM
    i = 1.0 / jnp.power(10000.0, i.astype(jnp.float32))
    mat = jnp.outer(pos, i)
    mat = mat.at[:, ::2].set(jnp.sin(mat[:, ::2]))
    mat = mat.at[:, 1::2].set(jnp.cos(mat[:, 1::2]))
    return mat


@functools.partial(
    pl.kernel,
    mesh=plsc.VectorSubcoreMesh(core_axis_name="c", subcore_axis_name="s"),
    out_type=jax.ShapeDtypeStruct((ROWS, DIM), jnp.float32),
    scratch_types=[
        pltpu.VMEM((SEQ, CHUNK), jnp.int32),
        pltpu.VMEM((SEQ, DIM), jnp.float32),
        pltpu.VMEM((NBUF, CHUNK, DIM), jnp.float32),
        pltpu.VMEM((NBUF, CHUNK), jnp.int32),
        [pltpu.SemaphoreType.DMA] * NBUF,
        [pltpu.SemaphoreType.DMA] * NBUF,
        [pltpu.SemaphoreType.DMA] * 2,
    ],
)
def _embed_kernel(xt_hbm, table_hbm, pe_hbm, out_hbm,
                  idx_v, pe_v, rows_v, didx_v, gsem, ssem, stage_sem):
    wid = lax.axis_index("s") * NUM_CORES + lax.axis_index("c")
    b_base = wid * BPW

    idx_cp = pltpu.async_copy(xt_hbm.at[:, pl.ds(b_base, CHUNK)], idx_v,
                              stage_sem[0])
    pe_cp = pltpu.async_copy(pe_hbm, pe_v, stage_sem[1])
    idx_cp.wait()

    ramp = [lax.iota(jnp.int32, LANES) * SEQ + (b_base + j * LANES) * SEQ
            for j in range(CHUNK // LANES)]

    def start_gather(s, buf):
        pltpu.async_copy(table_hbm.at[idx_v.at[s]], rows_v.at[buf], gsem[buf])

    def wait_gather(s, buf):
        pltpu.make_async_copy(table_hbm.at[idx_v.at[s]], rows_v.at[buf],
                              gsem[buf]).wait()

    def start_scatter(s, buf):
        for j in range(CHUNK // LANES):
            didx_v[buf, pl.ds(j * LANES, LANES)] = ramp[j] + s
        pltpu.async_copy(rows_v.at[buf], out_hbm.at[didx_v.at[buf]], ssem[buf])

    def wait_scatter(buf):
        pltpu.make_async_copy(rows_v.at[buf], out_hbm.at[didx_v.at[buf]],
                              ssem[buf]).wait()

    def add_pe(s, buf):
        rows = rows_v.at[buf]
        pe_row = [pe_v[s, pl.ds(j * LANES, LANES)] for j in range(DIM // LANES)]

        @pl.loop(0, CHUNK, unroll=4)
        def _row(r):
            for j in range(DIM // LANES):
                plsc.addupdate(rows.at[r, pl.ds(j * LANES, LANES)], pe_row[j])

    for k in range(NBUF - 1):
        start_gather(k, k)
    pe_cp.wait()

    @pl.loop(0, SEQ, step=NBUF)
    def _outer(s0):
        for b in range(NBUF):
            s = s0 + b
            wait_gather(s, b)

            fb = (b + NBUF - 1) % NBUF
            fs = s + NBUF - 1

            @pl.when(fs < SEQ)
            def _():
                @pl.when(s > 0)
                def _():
                    wait_scatter(fb)

                start_gather(fs, fb)

            add_pe(s, b)
            start_scatter(s, b)

    for s in range(SEQ - NBUF, SEQ):
        wait_scatter(s % NBUF)


def kernel(x, table):
    pe = _positional_encoding()
    xt = x.astype(jnp.int32).T
    out = _embed_kernel(xt, table, pe)
    return out.reshape(BATCH, SEQ, DIM)


# speedup vs baseline: 9.5080x; 1.0010x over previous
import functools

import jax
import jax.numpy as jnp
from jax import lax
from jax.experimental import pallas as pl
from jax.experimental.pallas import tpu as pltpu
from jax.experimental.pallas import tpu_sc as plsc

BATCH = 4096
SEQ = 200
DIM = 128
LANES = 16

NUM_CORES = 2
NUM_SUBCORES = 16
NUM_WORKERS = NUM_CORES * NUM_SUBCORES

BPW = BATCH // NUM_WORKERS
CHUNK = BPW
NBUF = 4
ROWS = BATCH * SEQ


def _positional_encoding():
    pos = jnp.arange(0, SEQ, dtype=jnp.float32)
    i = jnp.arange(0, DIM)
    i = 2 * (i // 2) / DIM
    i = 1.0 / jnp.power(10000.0, i.astype(jnp.float32))
    mat = jnp.outer(pos, i)
    mat = mat.at[:, ::2].set(jnp.sin(mat[:, ::2]))
    mat = mat.at[:, 1::2].set(jnp.cos(mat[:, 1::2]))
    return mat


@functools.cache
def _build_embed_kernel():
    @functools.partial(
        pl.kernel,
        mesh=plsc.VectorSubcoreMesh(core_axis_name="c", subcore_axis_name="s",
                                    num_cores=NUM_CORES,
                                    num_subcores=NUM_SUBCORES),
        out_type=jax.ShapeDtypeStruct((ROWS, DIM), jnp.float32),
        scratch_types=[
            pltpu.VMEM((SEQ, CHUNK), jnp.int32),
            pltpu.VMEM((SEQ, DIM), jnp.float32),
            pltpu.VMEM((NBUF, CHUNK, DIM), jnp.float32),
            pltpu.VMEM((NBUF, CHUNK), jnp.int32),
            [pltpu.SemaphoreType.DMA] * NBUF,
            [pltpu.SemaphoreType.DMA] * NBUF,
            [pltpu.SemaphoreType.DMA] * 2,
        ],
    )
    def _embed_kernel(xt_hbm, table_hbm, pe_hbm, out_hbm,
                      idx_v, pe_v, rows_v, didx_v, gsem, ssem, stage_sem):
        wid = lax.axis_index("s") * NUM_CORES + lax.axis_index("c")
        b_base = wid * BPW

        idx_cp = pltpu.async_copy(xt_hbm.at[:, pl.ds(b_base, CHUNK)], idx_v,
                                  stage_sem[0])
        pe_cp = pltpu.async_copy(pe_hbm, pe_v, stage_sem[1])
        idx_cp.wait()

        ramp = [lax.iota(jnp.int32, LANES) * SEQ + (b_base + j * LANES) * SEQ
                for j in range(CHUNK // LANES)]

        def start_gather(s, buf):
            pltpu.async_copy(table_hbm.at[idx_v.at[s]], rows_v.at[buf],
                             gsem[buf])

        def wait_gather(s, buf):
            pltpu.make_async_copy(table_hbm.at[idx_v.at[s]], rows_v.at[buf],
                                  gsem[buf]).wait()

        def start_scatter(s, buf):
            for j in range(CHUNK // LANES):
                didx_v[buf, pl.ds(j * LANES, LANES)] = ramp[j] + s
            pltpu.async_copy(rows_v.at[buf], out_hbm.at[didx_v.at[buf]],
                             ssem[buf])

        def wait_scatter(buf):
            pltpu.make_async_copy(rows_v.at[buf], out_hbm.at[didx_v.at[buf]],
                                  ssem[buf]).wait()

        def add_pe(s, buf):
            rows = rows_v.at[buf]
            pe_row = [pe_v[s, pl.ds(j * LANES, LANES)]
                      for j in range(DIM // LANES)]

            @pl.loop(0, CHUNK, unroll=4)
            def _row(r):
                for j in range(DIM // LANES):
                    plsc.addupdate(rows.at[r, pl.ds(j * LANES, LANES)],
                                   pe_row[j])

        for k in range(NBUF - 1):
            start_gather(k, k)
        pe_cp.wait()

        @pl.loop(0, SEQ, step=NBUF)
        def _outer(s0):
            for b in range(NBUF):
                s = s0 + b
                wait_gather(s, b)

                fb = (b + NBUF - 1) % NBUF
                fs = s + NBUF - 1

                @pl.when(fs < SEQ)
                def _():
                    @pl.when(s > 0)
                    def _():
                        wait_scatter(fb)

                    start_gather(fs, fb)

                add_pe(s, b)
                start_scatter(s, b)

        for s in range(SEQ - NBUF, SEQ):
            wait_scatter(s % NBUF)

    return _embed_kernel


def kernel(x, table):
    pe = _positional_encoding()
    xt = x.astype(jnp.int32).T
    out = _build_embed_kernel()(xt, table, pe)
    return out.reshape(BATCH, SEQ, DIM)
